# Initial kernel scaffold; baseline (speedup 1.0000x reference)
import jax
import jax.numpy as jnp
from jax.experimental import pallas as pl


def kernel(pred_boxes, pred_classes, anchors, gt_boxes, gt_classes):
    raise NotImplementedError("write your pallas kernel here")


# speedup vs baseline: 2.1746x; 2.1746x over previous
import functools

import jax
import jax.numpy as jnp
from jax import lax
from jax.experimental import pallas as pl
from jax.experimental.pallas import tpu as pltpu

B, A, O, C = 8, 32768, 100, 81
G = 104
T = 1024
THRESH = 0.4


def _body(pc_ref, anc_ref, gtb_ref, lab_ref, acc_ref):
    b = pl.program_id(0)
    a = pl.program_id(1)

    anc = anc_ref[0]
    acx, acy, aw, ah = anc[0:1, :], anc[1:2, :], anc[2:3, :], anc[3:4, :]
    ax1 = acx - aw * 0.5
    ay1 = acy - ah * 0.5
    ax2 = acx + aw * 0.5
    ay2 = acy + ah * 0.5
    area_a = (ax2 - ax1) * (ay2 - ay1)

    gtb = gtb_ref[0]
    gcx, gcy, gw, gh = gtb[:, 0:1], gtb[:, 1:2], gtb[:, 2:3], gtb[:, 3:4]
    gx1 = gcx - gw * 0.5
    gy1 = gcy - gh * 0.5
    gx2 = gcx + gw * 0.5
    gy2 = gcy + gh * 0.5
    area_g = (gx2 - gx1) * (gy2 - gy1)

    x1 = jnp.maximum(gx1, ax1)
    y1 = jnp.maximum(gy1, ay1)
    x2 = jnp.minimum(gx2, ax2)
    y2 = jnp.minimum(gy2, ay2)
    inter = jnp.clip(x2 - x1, 0.0) * jnp.clip(y2 - y1, 0.0)
    iou = inter / (area_g + area_a - inter + 1e-9)
    posf = (iou > THRESH).astype(jnp.bfloat16)

    lab = lab_ref[0]
    cio = lax.broadcasted_iota(jnp.int32, (G, C), 1)
    oh = (lab == cio).astype(jnp.bfloat16)

    w = lax.dot_general(
        posf, oh, (((0,), (0,)), ((), ())), preferred_element_type=jnp.float32
    )

    x = pc_ref[0]
    m = jnp.max(x, axis=1, keepdims=True)
    lse = m + jnp.log(jnp.sum(jnp.exp(x - m), axis=1, keepdims=True))

    pcnt = jnp.sum(w, axis=1, keepdims=True)
    negf = (pcnt == 0.0).astype(jnp.float32)

    s_logp = (
        jnp.sum(w * x)
        + jnp.sum(negf * x[:, 0:1])
        - jnp.sum((pcnt + negf) * lse)
    )
    n_pos = jnp.sum(pcnt)
    n_neg = jnp.sum(negf)

    rio = lax.broadcasted_iota(jnp.int32, (8, 128), 0)
    contrib = jnp.where(
        rio == 0, s_logp, jnp.where(rio == 1, n_pos, jnp.where(rio == 2, n_neg, 0.0))
    )

    @pl.when((b == 0) & (a == 0))
    def _init():
        acc_ref[...] = contrib

    @pl.when((b > 0) | (a > 0))
    def _accum():
        acc_ref[...] = acc_ref[...] + contrib


@jax.jit
def kernel(pred_boxes, pred_classes, anchors, gt_boxes, gt_classes):
    del pred_boxes
    ancT = jnp.transpose(anchors[..., 2:], (0, 2, 1))
    gtb = jnp.pad(gt_boxes, ((0, 0), (0, G - O), (0, 0)), constant_values=-10.0)
    lab = jnp.pad(gt_classes.astype(jnp.int32) + 1, ((0, 0), (0, G - O)))[..., None]

    acc = pl.pallas_call(
        _body,
        grid=(B, A // T),
        in_specs=[
            pl.BlockSpec((1, T, C), lambda b, a: (b, a, 0)),
            pl.BlockSpec((1, 4, T), lambda b, a: (b, 0, a)),
            pl.BlockSpec((1, G, 4), lambda b, a: (b, 0, 0)),
            pl.BlockSpec((1, G, 1), lambda b, a: (b, 0, 0)),
        ],
        out_specs=pl.BlockSpec((8, 128), lambda b, a: (0, 0)),
        out_shape=jax.ShapeDtypeStruct((8, 128), jnp.float32),
        compiler_params=pltpu.CompilerParams(
            dimension_semantics=("arbitrary", "arbitrary"),
        ),
    )(pred_classes, ancT, gtb, lab)

    s_logp = acc[0, 0]
    n_pos = acc[1, 0]
    n_neg = acc[2, 0]
    return -s_logp / (n_pos + n_neg)
